# Initial kernel scaffold; baseline (speedup 1.0000x reference)
import jax
import jax.numpy as jnp
from jax.experimental import pallas as pl


def kernel(x, edge_index, W1, b1, W2, b2, W3, b3, W4, b4):
    raise NotImplementedError("write your pallas kernel here")


# speedup vs baseline: 64.5816x; 64.5816x over previous
import functools

import jax
import jax.numpy as jnp
from jax import lax
from jax.experimental import pallas as pl
from jax.experimental.pallas import tpu as pltpu
from jax.experimental.pallas import tpu_sc as plsc

N = 10000
E = 320000
D = 128
F = 16

NC = 2
NS = 16
L = 16
NW = NC * NS

N_PAD = 10240
SLICE = N_PAD // NS
CHUNK = 128
ROWS_PER_W = 80
E_PAD = NW * ROWS_PER_W * CHUNK
EROWS = E_PAD // CHUNK

def _deg_body(dst_hbm, deg_out, idx_v, ones_v, z_v, deg_sh):
    cid = lax.axis_index("c")
    sid = lax.axis_index("s")
    wid = sid * NC + cid
    for j in range(CHUNK // L):
        ones_v[pl.ds(j * L, L)] = jnp.ones((L,), jnp.float32)
    for j in range(SLICE // L):
        z_v[pl.ds(j * L, L)] = jnp.zeros((L,), jnp.float32)
    pltpu.sync_copy(z_v, deg_sh.at[pl.ds(sid * SLICE, SLICE)])
    pltpu.sync_copy(dst_hbm.at[pl.ds(wid * ROWS_PER_W, ROWS_PER_W)], idx_v)
    plsc.subcore_barrier()

    @pl.loop(0, ROWS_PER_W)
    def _(j):
        pltpu.sync_copy(ones_v, deg_sh.at[idx_v.at[j]], add=True)

    plsc.subcore_barrier()
    pltpu.sync_copy(
        deg_sh.at[pl.ds(sid * SLICE, SLICE)],
        deg_out.at[pl.ds(cid * N_PAD + sid * SLICE, SLICE)],
    )


def _scat_body(src_hbm, dst_hbm, hn_hbm, s_out, si_v, di_v, rows_v, z_v,
               hn_sh, s_sh, sem):
    cid = lax.axis_index("c")
    sid = lax.axis_index("s")
    wid = sid * NC + cid
    pltpu.sync_copy(
        hn_hbm.at[pl.ds(sid * SLICE, SLICE)],
        hn_sh.at[pl.ds(sid * SLICE, SLICE)],
    )
    for j in range(CHUNK):
        z_v[j, :] = jnp.zeros((L,), jnp.float32)
    for k in range(SLICE // CHUNK):
        pltpu.sync_copy(z_v, s_sh.at[pl.ds(sid * SLICE + k * CHUNK, CHUNK)])
    pltpu.sync_copy(src_hbm.at[pl.ds(wid * ROWS_PER_W, ROWS_PER_W)], si_v)
    pltpu.sync_copy(dst_hbm.at[pl.ds(wid * ROWS_PER_W, ROWS_PER_W)], di_v)
    plsc.subcore_barrier()

    @pl.loop(0, ROWS_PER_W)
    def _(j):
        pltpu.async_copy(hn_sh.at[si_v.at[j]], rows_v, sem).wait()
        pltpu.sync_copy(rows_v, s_sh.at[di_v.at[j]], add=True)

    plsc.subcore_barrier()
    pltpu.sync_copy(
        s_sh.at[pl.ds(sid * SLICE, SLICE)],
        s_out.at[pl.ds(cid * N_PAD + sid * SLICE, SLICE)],
    )


@functools.lru_cache(maxsize=None)
def _sc_kernels():
    mesh = plsc.VectorSubcoreMesh(
        core_axis_name="c", subcore_axis_name="s",
        num_cores=NC, num_subcores=NS,
    )
    params = pltpu.CompilerParams(use_tc_tiling_on_sc=False)
    deg_kernel = pl.kernel(
        _deg_body,
        out_type=jax.ShapeDtypeStruct((NC * N_PAD,), jnp.float32),
        mesh=mesh,
        compiler_params=params,
        scratch_types=[
            pltpu.VMEM((ROWS_PER_W, CHUNK), jnp.int32),
            pltpu.VMEM((CHUNK,), jnp.float32),
            pltpu.VMEM((SLICE,), jnp.float32),
            pltpu.VMEM_SHARED((N_PAD,), jnp.float32),
        ],
    )
    scat_kernel = pl.kernel(
        _scat_body,
        out_type=jax.ShapeDtypeStruct((NC * N_PAD, F), jnp.float32),
        mesh=mesh,
        compiler_params=params,
        scratch_types=[
            pltpu.VMEM((ROWS_PER_W, CHUNK), jnp.int32),
            pltpu.VMEM((ROWS_PER_W, CHUNK), jnp.int32),
            pltpu.VMEM((CHUNK, F), jnp.float32),
            pltpu.VMEM((CHUNK, F), jnp.float32),
            pltpu.VMEM_SHARED((N_PAD, F), jnp.float32),
            pltpu.VMEM_SHARED((N_PAD, F), jnp.float32),
            pltpu.SemaphoreType.DMA,
        ],
    )
    return deg_kernel, scat_kernel


BLK = 1024
GRID = N_PAD // BLK


def _hn_body(x_ref, w1_ref, degp_ref, hn_ref, dinv_ref):
    deg = degp_ref[0, :] + degp_ref[1, :] + 1.0
    dinv = lax.rsqrt(deg)
    h = jnp.dot(x_ref[...], w1_ref[...], preferred_element_type=jnp.float32)
    hn_ref[...] = h * dinv[:, None]
    dinv_ref[...] = dinv[None, :]


def _hn_call(x_p, W1, degp):
    return pl.pallas_call(
        _hn_body,
        grid=(GRID,),
        in_specs=[
            pl.BlockSpec((BLK, D), lambda i: (i, 0)),
            pl.BlockSpec((D, F), lambda i: (0, 0)),
            pl.BlockSpec((NC, BLK), lambda i: (0, i)),
        ],
        out_specs=[
            pl.BlockSpec((BLK, F), lambda i: (i, 0)),
            pl.BlockSpec((1, BLK), lambda i: (0, i)),
        ],
        out_shape=[
            jax.ShapeDtypeStruct((N_PAD, F), jnp.float32),
            jax.ShapeDtypeStruct((1, N_PAD), jnp.float32),
        ],
    )(x_p, W1, degp)


def _mlp_body(sp_ref, hn_ref, dinv_ref, b1_ref, w2_ref, b2_ref, w3_ref,
              b3_ref, w4_ref, b4_ref, out_ref):
    s = sp_ref[0] + sp_ref[1] + hn_ref[...]
    dinv = dinv_ref[0, :]
    a = jnp.maximum(s * dinv[:, None] + b1_ref[...], 0.0)
    a = jnp.maximum(
        jnp.dot(a, w2_ref[...], preferred_element_type=jnp.float32)
        + b2_ref[...], 0.0)
    a = jnp.maximum(
        jnp.dot(a, w3_ref[...], preferred_element_type=jnp.float32)
        + b3_ref[...], 0.0)
    z = (jnp.dot(a, w4_ref[...], preferred_element_type=jnp.float32)
         + b4_ref[...])
    out_ref[...] = jax.nn.sigmoid(z)


def _mlp_call(sp, hn, dinv, b1, W2, b2, W3, b3, W4, b4):
    return pl.pallas_call(
        _mlp_body,
        grid=(GRID,),
        in_specs=[
            pl.BlockSpec((NC, BLK, F), lambda i: (0, i, 0)),
            pl.BlockSpec((BLK, F), lambda i: (i, 0)),
            pl.BlockSpec((1, BLK), lambda i: (0, i)),
            pl.BlockSpec((1, F), lambda i: (0, 0)),
            pl.BlockSpec((F, 16), lambda i: (0, 0)),
            pl.BlockSpec((1, 16), lambda i: (0, 0)),
            pl.BlockSpec((16, 32), lambda i: (0, 0)),
            pl.BlockSpec((1, 32), lambda i: (0, 0)),
            pl.BlockSpec((32, 1), lambda i: (0, 0)),
            pl.BlockSpec((1, 1), lambda i: (0, 0)),
        ],
        out_specs=pl.BlockSpec((BLK, 1), lambda i: (i, 0)),
        out_shape=jax.ShapeDtypeStruct((N_PAD, 1), jnp.float32),
    )(sp, hn, dinv, b1, W2, b2, W3, b3, W4, b4)


def kernel(x, edge_index, W1, b1, W2, b2, W3, b3, W4, b4):
    src = edge_index[0]
    dst = edge_index[1]
    npad_e = E_PAD - E
    pad_ar = jnp.arange(npad_e, dtype=jnp.int32)
    src_p = jnp.concatenate([src, pad_ar % N]).reshape(EROWS, CHUNK)
    dst_p = jnp.concatenate([dst, N + pad_ar % (N_PAD - N)]).reshape(EROWS, CHUNK)

    deg_kernel, scat_kernel = _sc_kernels()
    degp = deg_kernel(dst_p).reshape(NC, N_PAD)
    x_p = jnp.pad(x, ((0, N_PAD - N), (0, 0)))
    hn, dinv = _hn_call(x_p, W1, degp)
    sp = scat_kernel(src_p, dst_p, hn).reshape(NC, N_PAD, F)
    out = _mlp_call(sp, hn, dinv, b1.reshape(1, F), W2, b2.reshape(1, 16),
                    W3, b3.reshape(1, 32), W4, b4.reshape(1, 1))
    return out[:N]


# speedup vs baseline: 70.7725x; 1.0959x over previous
import functools

import jax
import jax.numpy as jnp
from jax import lax
from jax.experimental import pallas as pl
from jax.experimental.pallas import tpu as pltpu
from jax.experimental.pallas import tpu_sc as plsc

N = 10000
E = 320000
D = 128
F = 16

NC = 2
NS = 16
L = 16
NW = NC * NS

N_PAD = 10240
SLICE = N_PAD // NS
CHUNK = 128
ROWS_PER_W = 80
E_PAD = NW * ROWS_PER_W * CHUNK
EROWS = E_PAD // CHUNK

DEG_DEPTH = 16


def _deg_body(dst_hbm, deg_out, idx_v, ones_v, z_v, deg_sh, sem):
    cid = lax.axis_index("c")
    sid = lax.axis_index("s")
    wid = sid * NC + cid
    for j in range(CHUNK // L):
        ones_v[pl.ds(j * L, L)] = jnp.ones((L,), jnp.float32)
    for j in range(SLICE // L):
        z_v[pl.ds(j * L, L)] = jnp.zeros((L,), jnp.float32)
    pltpu.sync_copy(z_v, deg_sh.at[pl.ds(sid * SLICE, SLICE)])
    pltpu.sync_copy(dst_hbm.at[pl.ds(wid * ROWS_PER_W, ROWS_PER_W)], idx_v)
    plsc.subcore_barrier()

    for j in range(DEG_DEPTH):
        pltpu.async_copy(ones_v, deg_sh.at[idx_v.at[j]], sem, add=True)

    @pl.loop(0, ROWS_PER_W - DEG_DEPTH)
    def _(g):
        pltpu.make_async_copy(ones_v, deg_sh.at[pl.ds(0, CHUNK)], sem).wait()
        pltpu.async_copy(ones_v, deg_sh.at[idx_v.at[g + DEG_DEPTH]], sem,
                         add=True)

    for j in range(DEG_DEPTH):
        pltpu.make_async_copy(ones_v, deg_sh.at[pl.ds(0, CHUNK)], sem).wait()

    plsc.subcore_barrier()
    pltpu.sync_copy(
        deg_sh.at[pl.ds(sid * SLICE, SLICE)],
        deg_out.at[pl.ds(cid * N_PAD + sid * SLICE, SLICE)],
    )


NBUF = 4


def _scat_body(src_hbm, dst_hbm, hn_hbm, s_out, si_v, di_v, rows_v, z_v,
               hn_sh, s_sh, *gsems):
    cid = lax.axis_index("c")
    sid = lax.axis_index("s")
    wid = sid * NC + cid
    pltpu.sync_copy(
        hn_hbm.at[pl.ds(sid * SLICE, SLICE)],
        hn_sh.at[pl.ds(sid * SLICE, SLICE)],
    )
    for j in range(CHUNK):
        z_v[j, :] = jnp.zeros((L,), jnp.float32)
    for k in range(SLICE // CHUNK):
        pltpu.sync_copy(z_v, s_sh.at[pl.ds(sid * SLICE + k * CHUNK, CHUNK)])
    pltpu.sync_copy(src_hbm.at[pl.ds(wid * ROWS_PER_W, ROWS_PER_W)], si_v)
    pltpu.sync_copy(dst_hbm.at[pl.ds(wid * ROWS_PER_W, ROWS_PER_W)], di_v)
    plsc.subcore_barrier()

    for b in range(NBUF):
        pltpu.async_copy(hn_sh.at[si_v.at[b]], rows_v.at[b], gsems[b])

    @pl.loop(0, ROWS_PER_W // NBUF)
    def _(g):
        for b in range(NBUF):
            j = g * NBUF + b
            pltpu.make_async_copy(hn_sh.at[pl.ds(0, CHUNK)], rows_v.at[b],
                                  gsems[b]).wait()
            pltpu.sync_copy(rows_v.at[b], s_sh.at[di_v.at[j]], add=True)

            @pl.when(g < ROWS_PER_W // NBUF - 1)
            def _():
                pltpu.async_copy(hn_sh.at[si_v.at[j + NBUF]], rows_v.at[b],
                                 gsems[b])

    plsc.subcore_barrier()
    pltpu.sync_copy(
        s_sh.at[pl.ds(sid * SLICE, SLICE)],
        s_out.at[pl.ds(cid * N_PAD + sid * SLICE, SLICE)],
    )


@functools.lru_cache(maxsize=None)
def _sc_kernels():
    mesh = plsc.VectorSubcoreMesh(
        core_axis_name="c", subcore_axis_name="s",
        num_cores=NC, num_subcores=NS,
    )
    params = pltpu.CompilerParams(use_tc_tiling_on_sc=False)
    deg_kernel = pl.kernel(
        _deg_body,
        out_type=jax.ShapeDtypeStruct((NC * N_PAD,), jnp.float32),
        mesh=mesh,
        compiler_params=params,
        scratch_types=[
            pltpu.VMEM((ROWS_PER_W, CHUNK), jnp.int32),
            pltpu.VMEM((CHUNK,), jnp.float32),
            pltpu.VMEM((SLICE,), jnp.float32),
            pltpu.VMEM_SHARED((N_PAD,), jnp.float32),
            pltpu.SemaphoreType.DMA,
        ],
    )
    scat_kernel = pl.kernel(
        _scat_body,
        out_type=jax.ShapeDtypeStruct((NC * N_PAD, F), jnp.float32),
        mesh=mesh,
        compiler_params=params,
        scratch_types=[
            pltpu.VMEM((ROWS_PER_W, CHUNK), jnp.int32),
            pltpu.VMEM((ROWS_PER_W, CHUNK), jnp.int32),
            pltpu.VMEM((NBUF, CHUNK, F), jnp.float32),
            pltpu.VMEM((CHUNK, F), jnp.float32),
            pltpu.VMEM_SHARED((N_PAD, F), jnp.float32),
            pltpu.VMEM_SHARED((N_PAD, F), jnp.float32),
        ] + [pltpu.SemaphoreType.DMA] * NBUF,
    )
    return deg_kernel, scat_kernel


BLK = 1024
GRID = N_PAD // BLK


def _mm_body(x_ref, w1_ref, h_ref):
    h_ref[...] = jnp.dot(x_ref[...], w1_ref[...],
                         preferred_element_type=jnp.float32)


def _mm_call(x_p, W1):
    return pl.pallas_call(
        _mm_body,
        grid=(GRID,),
        in_specs=[
            pl.BlockSpec((BLK, D), lambda i: (i, 0)),
            pl.BlockSpec((D, F), lambda i: (0, 0)),
        ],
        out_specs=pl.BlockSpec((BLK, F), lambda i: (i, 0)),
        out_shape=jax.ShapeDtypeStruct((N_PAD, F), jnp.float32),
    )(x_p, W1)


def _hn_body(h_ref, deg0_ref, deg1_ref, hn_ref, dinv_ref):
    deg = deg0_ref[...] + deg1_ref[...] + 1.0
    dinv = lax.rsqrt(deg)
    hn_ref[...] = h_ref[...] * dinv[:, None]
    dinv_ref[...] = dinv[None, :]


def _hn_call(h, degp_flat):
    return pl.pallas_call(
        _hn_body,
        grid=(GRID,),
        in_specs=[
            pl.BlockSpec((BLK, F), lambda i: (i, 0)),
            pl.BlockSpec((BLK,), lambda i: (i,)),
            pl.BlockSpec((BLK,), lambda i: (i + GRID,)),
        ],
        out_specs=[
            pl.BlockSpec((BLK, F), lambda i: (i, 0)),
            pl.BlockSpec((1, BLK), lambda i: (0, i)),
        ],
        out_shape=[
            jax.ShapeDtypeStruct((N_PAD, F), jnp.float32),
            jax.ShapeDtypeStruct((1, N_PAD), jnp.float32),
        ],
    )(h, degp_flat, degp_flat)


def _mlp_body(sp_ref, hn_ref, dinv_ref, b1_ref, w2_ref, b2_ref, w3_ref,
              b3_ref, w4_ref, b4_ref, out_ref):
    s = sp_ref[0] + sp_ref[1] + hn_ref[...]
    dinv = dinv_ref[0, :]
    a = jnp.maximum(s * dinv[:, None] + b1_ref[...], 0.0)
    a = jnp.maximum(
        jnp.dot(a, w2_ref[...], preferred_element_type=jnp.float32)
        + b2_ref[...], 0.0)
    a = jnp.maximum(
        jnp.dot(a, w3_ref[...], preferred_element_type=jnp.float32)
        + b3_ref[...], 0.0)
    z = (jnp.dot(a, w4_ref[...], preferred_element_type=jnp.float32)
         + b4_ref[...])
    out_ref[...] = jax.nn.sigmoid(z)


def _mlp_call(sp, hn, dinv, b1, W2, b2, W3, b3, W4, b4):
    return pl.pallas_call(
        _mlp_body,
        grid=(GRID,),
        in_specs=[
            pl.BlockSpec((NC, BLK, F), lambda i: (0, i, 0)),
            pl.BlockSpec((BLK, F), lambda i: (i, 0)),
            pl.BlockSpec((1, BLK), lambda i: (0, i)),
            pl.BlockSpec((1, F), lambda i: (0, 0)),
            pl.BlockSpec((F, 16), lambda i: (0, 0)),
            pl.BlockSpec((1, 16), lambda i: (0, 0)),
            pl.BlockSpec((16, 32), lambda i: (0, 0)),
            pl.BlockSpec((1, 32), lambda i: (0, 0)),
            pl.BlockSpec((32, 1), lambda i: (0, 0)),
            pl.BlockSpec((1, 1), lambda i: (0, 0)),
        ],
        out_specs=pl.BlockSpec((BLK, 1), lambda i: (i, 0)),
        out_shape=jax.ShapeDtypeStruct((N_PAD, 1), jnp.float32),
    )(sp, hn, dinv, b1, W2, b2, W3, b3, W4, b4)


def kernel(x, edge_index, W1, b1, W2, b2, W3, b3, W4, b4):
    src = edge_index[0]
    dst = edge_index[1]
    npad_e = E_PAD - E
    pad_ar = jnp.arange(npad_e, dtype=jnp.int32)
    src_p = jnp.concatenate([src, pad_ar % N]).reshape(EROWS, CHUNK)
    dst_p = jnp.concatenate([dst, N + pad_ar % (N_PAD - N)]).reshape(EROWS, CHUNK)

    deg_kernel, scat_kernel = _sc_kernels()
    degp_flat = deg_kernel(dst_p)
    x_p = jnp.pad(x, ((0, N_PAD - N), (0, 0)))
    h = _mm_call(x_p, W1)
    hn, dinv = _hn_call(h, degp_flat)
    sp = scat_kernel(src_p, dst_p, hn).reshape(NC, N_PAD, F)
    out = _mlp_call(sp, hn, dinv, b1.reshape(1, F), W2, b2.reshape(1, 16),
                    W3, b3.reshape(1, 32), W4, b4.reshape(1, 1))
    return out[:N]


# speedup vs baseline: 80.9696x; 1.1441x over previous
import functools

import jax
import jax.numpy as jnp
from jax import lax
from jax.experimental import pallas as pl
from jax.experimental.pallas import tpu as pltpu
from jax.experimental.pallas import tpu_sc as plsc

N = 10000
E = 320000
D = 128
F = 16

NC = 2
NS = 16
L = 16
NW = NC * NS

CHUNK = 128
EROWS = E // CHUNK
NR = EROWS // NW
NLEFT = EROWS - NR * NW
NRMAX = NR + 1

NP = 10240
SLICE_N = NP // NS

DEG_DEPTH = 16
NBUF = 4


def _deg_body(ei_hbm, deg_out, idx_v, ones_v, z_v, deg_sh, sem):
    cid = lax.axis_index("c")
    sid = lax.axis_index("s")
    wid = sid * NC + cid
    nrows = NR + jnp.where(wid < NLEFT, 1, 0)
    for j in range(CHUNK // L):
        ones_v[pl.ds(j * L, L)] = jnp.ones((L,), jnp.float32)
    for j in range(SLICE_N // L):
        z_v[pl.ds(j * L, L)] = jnp.zeros((L,), jnp.float32)
    pltpu.sync_copy(z_v, deg_sh.at[pl.ds(sid * SLICE_N, SLICE_N)])
    pltpu.sync_copy(ei_hbm.at[pl.ds(wid * NR, NR), 1], idx_v.at[pl.ds(0, NR)])

    @pl.when(wid < NLEFT)
    def _():
        pltpu.sync_copy(ei_hbm.at[NW * NR + wid, 1], idx_v.at[NR])

    plsc.subcore_barrier()

    for j in range(DEG_DEPTH):
        pltpu.async_copy(ones_v, deg_sh.at[idx_v.at[j]], sem, add=True)

    @pl.loop(0, NRMAX - DEG_DEPTH)
    def _(g):
        @pl.when(g + DEG_DEPTH < nrows)
        def _():
            pltpu.make_async_copy(ones_v, deg_sh.at[pl.ds(0, CHUNK)],
                                  sem).wait()
            pltpu.async_copy(ones_v, deg_sh.at[idx_v.at[g + DEG_DEPTH]], sem,
                             add=True)

    for j in range(DEG_DEPTH):
        pltpu.make_async_copy(ones_v, deg_sh.at[pl.ds(0, CHUNK)], sem).wait()

    plsc.subcore_barrier()
    pltpu.sync_copy(
        deg_sh.at[pl.ds(sid * SLICE_N, SLICE_N)],
        deg_out.at[pl.ds(cid * NP + sid * SLICE_N, SLICE_N)],
    )


def _scat_body(ei_hbm, hn_hbm, s_out, si_v, di_v, rows_v, z_v,
               hn_sh, s_sh, *gsems):
    cid = lax.axis_index("c")
    sid = lax.axis_index("s")
    wid = sid * NC + cid
    nrows = NR + jnp.where(wid < NLEFT, 1, 0)
    pltpu.sync_copy(
        hn_hbm.at[pl.ds(sid * SLICE_N, SLICE_N)],
        hn_sh.at[pl.ds(sid * SLICE_N, SLICE_N)],
    )
    for j in range(CHUNK):
        z_v[j, :] = jnp.zeros((L,), jnp.float32)
    for k in range(SLICE_N // CHUNK):
        pltpu.sync_copy(
            z_v, s_sh.at[pl.ds(sid * SLICE_N + k * CHUNK, CHUNK)])
    pltpu.sync_copy(ei_hbm.at[pl.ds(wid * NR, NR), 0], si_v.at[pl.ds(0, NR)])
    pltpu.sync_copy(ei_hbm.at[pl.ds(wid * NR, NR), 1], di_v.at[pl.ds(0, NR)])

    @pl.when(wid < NLEFT)
    def _():
        pltpu.sync_copy(ei_hbm.at[NW * NR + wid, 0], si_v.at[NR])
        pltpu.sync_copy(ei_hbm.at[NW * NR + wid, 1], di_v.at[NR])

    plsc.subcore_barrier()

    for b in range(NBUF):
        pltpu.async_copy(hn_sh.at[si_v.at[b]], rows_v.at[b], gsems[b])

    @pl.loop(0, NRMAX // NBUF + 1)
    def _(g):
        for b in range(NBUF):
            j = g * NBUF + b

            @pl.when(j < nrows)
            def _():
                pltpu.make_async_copy(hn_sh.at[pl.ds(0, CHUNK)], rows_v.at[b],
                                      gsems[b]).wait()
                pltpu.sync_copy(rows_v.at[b], s_sh.at[di_v.at[j]], add=True)

            @pl.when(j + NBUF < nrows)
            def _():
                pltpu.async_copy(hn_sh.at[si_v.at[j + NBUF]], rows_v.at[b],
                                 gsems[b])

    plsc.subcore_barrier()
    pltpu.sync_copy(
        s_sh.at[pl.ds(sid * SLICE_N, SLICE_N)],
        s_out.at[pl.ds(cid * NP + sid * SLICE_N, SLICE_N)],
    )


@functools.lru_cache(maxsize=None)
def _sc_kernels():
    mesh = plsc.VectorSubcoreMesh(
        core_axis_name="c", subcore_axis_name="s",
        num_cores=NC, num_subcores=NS,
    )
    params = pltpu.CompilerParams(use_tc_tiling_on_sc=False)
    deg_kernel = pl.kernel(
        _deg_body,
        out_type=jax.ShapeDtypeStruct((NC * NP,), jnp.float32),
        mesh=mesh,
        compiler_params=params,
        scratch_types=[
            pltpu.VMEM((NRMAX, CHUNK), jnp.int32),
            pltpu.VMEM((CHUNK,), jnp.float32),
            pltpu.VMEM((SLICE_N,), jnp.float32),
            pltpu.VMEM_SHARED((NP,), jnp.float32),
            pltpu.SemaphoreType.DMA,
        ],
    )
    scat_kernel = pl.kernel(
        _scat_body,
        out_type=jax.ShapeDtypeStruct((NC * NP, F), jnp.float32),
        mesh=mesh,
        compiler_params=params,
        scratch_types=[
            pltpu.VMEM((NRMAX, CHUNK), jnp.int32),
            pltpu.VMEM((NRMAX, CHUNK), jnp.int32),
            pltpu.VMEM((NBUF, CHUNK, F), jnp.float32),
            pltpu.VMEM((CHUNK, F), jnp.float32),
            pltpu.VMEM_SHARED((NP, F), jnp.float32),
            pltpu.VMEM_SHARED((NP, F), jnp.float32),
        ] + [pltpu.SemaphoreType.DMA] * NBUF,
    )
    return deg_kernel, scat_kernel


BLK = 1024
GRID = NP // BLK


def _mm_body(x_ref, w1_ref, h_ref):
    h_ref[...] = jnp.dot(x_ref[...], w1_ref[...],
                         preferred_element_type=jnp.float32)


def _mm_call(x_p, W1):
    return pl.pallas_call(
        _mm_body,
        grid=(GRID,),
        in_specs=[
            pl.BlockSpec((BLK, D), lambda i: (i, 0)),
            pl.BlockSpec((D, F), lambda i: (0, 0)),
        ],
        out_specs=pl.BlockSpec((BLK, F), lambda i: (i, 0)),
        out_shape=jax.ShapeDtypeStruct((NP, F), jnp.float32),
    )(x_p, W1)


def _hn_body(h_ref, degp_ref, hn_ref, dinv_ref):
    i = pl.program_id(0)
    deg = (degp_ref[pl.ds(i * BLK, BLK)]
           + degp_ref[pl.ds(NP + i * BLK, BLK)] + 1.0)
    dinv = lax.rsqrt(deg)
    hn_ref[...] = h_ref[...] * dinv[:, None]
    dinv_ref[0, pl.ds(i * BLK, BLK)] = dinv


def _hn_call(h, degp_flat):
    return pl.pallas_call(
        _hn_body,
        grid=(GRID,),
        in_specs=[
            pl.BlockSpec((BLK, F), lambda i: (i, 0)),
            pl.BlockSpec((NC * NP,), lambda i: (0,)),
        ],
        out_specs=[
            pl.BlockSpec((BLK, F), lambda i: (i, 0)),
            pl.BlockSpec((1, NP), lambda i: (0, 0)),
        ],
        out_shape=[
            jax.ShapeDtypeStruct((NP, F), jnp.float32),
            jax.ShapeDtypeStruct((1, NP), jnp.float32),
        ],
    )(h, degp_flat)


def _mlp_body(sp0_ref, sp1_ref, hn_ref, dinv_ref, b1_ref, w2_ref, b2_ref,
              w3_ref, b3_ref, w4_ref, b4_ref, out_ref):
    s = sp0_ref[...] + sp1_ref[...] + hn_ref[...]
    dinv = dinv_ref[0, pl.ds(pl.program_id(0) * BLK, BLK)]
    a = jnp.maximum(s * dinv[:, None] + b1_ref[...], 0.0)
    a = jnp.maximum(
        jnp.dot(a, w2_ref[...], preferred_element_type=jnp.float32)
        + b2_ref[...], 0.0)
    a = jnp.maximum(
        jnp.dot(a, w3_ref[...], preferred_element_type=jnp.float32)
        + b3_ref[...], 0.0)
    z = (jnp.dot(a, w4_ref[...], preferred_element_type=jnp.float32)
         + b4_ref[...])
    out_ref[...] = jax.nn.sigmoid(z)


def _mlp_call(sp, hn, dinv, b1, W2, b2, W3, b3, W4, b4):
    return pl.pallas_call(
        _mlp_body,
        grid=(GRID,),
        in_specs=[
            pl.BlockSpec((BLK, F), lambda i: (i, 0)),
            pl.BlockSpec((BLK, F), lambda i: (i + GRID, 0)),
            pl.BlockSpec((BLK, F), lambda i: (i, 0)),
            pl.BlockSpec((1, NP), lambda i: (0, 0)),
            pl.BlockSpec((1, F), lambda i: (0, 0)),
            pl.BlockSpec((F, 16), lambda i: (0, 0)),
            pl.BlockSpec((1, 16), lambda i: (0, 0)),
            pl.BlockSpec((16, 32), lambda i: (0, 0)),
            pl.BlockSpec((1, 32), lambda i: (0, 0)),
            pl.BlockSpec((32, 1), lambda i: (0, 0)),
            pl.BlockSpec((1, 1), lambda i: (0, 0)),
        ],
        out_specs=pl.BlockSpec((BLK, 1), lambda i: (i, 0)),
        out_shape=jax.ShapeDtypeStruct((NP, 1), jnp.float32),
    )(sp, sp, hn, dinv, b1, W2, b2, W3, b3, W4, b4)


def kernel(x, edge_index, W1, b1, W2, b2, W3, b3, W4, b4):
    ei3 = edge_index.reshape(2, EROWS, CHUNK).transpose(1, 0, 2)

    deg_kernel, scat_kernel = _sc_kernels()
    degp_flat = deg_kernel(ei3)
    x_p = jnp.pad(x, ((0, NP - N), (0, 0)))
    h = _mm_call(x_p, W1)
    hn, dinv = _hn_call(h, degp_flat)
    sp = scat_kernel(ei3, hn)
    out = _mlp_call(sp, hn, dinv, b1.reshape(1, F), W2, b2.reshape(1, 16),
                    W3, b3.reshape(1, 32), W4, b4.reshape(1, 1))
    return out[:N]


# speedup vs baseline: 82.5571x; 1.0196x over previous
import functools

import jax
import jax.numpy as jnp
from jax import lax
from jax.experimental import pallas as pl
from jax.experimental.pallas import tpu as pltpu
from jax.experimental.pallas import tpu_sc as plsc

N = 10000
E = 320000
D = 128
F = 16

NC = 2
NS = 16
L = 16
NW = NC * NS

CHUNK = 128
EROWS = E // CHUNK
NR = EROWS // NW
NLEFT = EROWS - NR * NW
NRMAX = NR + 1

NP = 10240
SLICE_N = NP // NS

DEG_DEPTH = 16
NBUF = 4


def _deg_body(ei_hbm, deg_out, idx_v, ones_v, z_v, deg_sh, sem):
    cid = lax.axis_index("c")
    sid = lax.axis_index("s")
    wid = sid * NC + cid
    nrows = NR + jnp.where(wid < NLEFT, 1, 0)
    for j in range(CHUNK // L):
        ones_v[pl.ds(j * L, L)] = jnp.ones((L,), jnp.float32)
    for j in range(SLICE_N // L):
        z_v[pl.ds(j * L, L)] = jnp.zeros((L,), jnp.float32)
    pltpu.sync_copy(z_v, deg_sh.at[pl.ds(sid * SLICE_N, SLICE_N)])
    pltpu.sync_copy(ei_hbm.at[pl.ds(wid * NR, NR), 1], idx_v.at[pl.ds(0, NR)])

    @pl.when(wid < NLEFT)
    def _():
        pltpu.sync_copy(ei_hbm.at[NW * NR + wid, 1], idx_v.at[NR])

    plsc.subcore_barrier()

    for j in range(DEG_DEPTH):
        pltpu.async_copy(ones_v, deg_sh.at[idx_v.at[j]], sem, add=True)

    @pl.loop(0, NRMAX - DEG_DEPTH)
    def _(g):
        @pl.when(g + DEG_DEPTH < nrows)
        def _():
            pltpu.make_async_copy(ones_v, deg_sh.at[pl.ds(0, CHUNK)],
                                  sem).wait()
            pltpu.async_copy(ones_v, deg_sh.at[idx_v.at[g + DEG_DEPTH]], sem,
                             add=True)

    for j in range(DEG_DEPTH):
        pltpu.make_async_copy(ones_v, deg_sh.at[pl.ds(0, CHUNK)], sem).wait()

    plsc.subcore_barrier()
    pltpu.sync_copy(
        deg_sh.at[pl.ds(sid * SLICE_N, SLICE_N)],
        deg_out.at[pl.ds(cid * NP + sid * SLICE_N, SLICE_N)],
    )


def _scat_body(ei_hbm, hn_hbm, s_out, si_v, di_v, rows_v, z_v,
               hn_sh, s_sh, *gsems):
    cid = lax.axis_index("c")
    sid = lax.axis_index("s")
    wid = sid * NC + cid
    nrows = NR + jnp.where(wid < NLEFT, 1, 0)
    pltpu.sync_copy(
        hn_hbm.at[pl.ds(sid * SLICE_N, SLICE_N), pl.ds(0, F)],
        hn_sh.at[pl.ds(sid * SLICE_N, SLICE_N)],
    )
    for j in range(CHUNK):
        z_v[j, :] = jnp.zeros((L,), jnp.float32)
    for k in range(SLICE_N // CHUNK):
        pltpu.sync_copy(
            z_v, s_sh.at[pl.ds(sid * SLICE_N + k * CHUNK, CHUNK)])
    pltpu.sync_copy(ei_hbm.at[pl.ds(wid * NR, NR), 0], si_v.at[pl.ds(0, NR)])
    pltpu.sync_copy(ei_hbm.at[pl.ds(wid * NR, NR), 1], di_v.at[pl.ds(0, NR)])

    @pl.when(wid < NLEFT)
    def _():
        pltpu.sync_copy(ei_hbm.at[NW * NR + wid, 0], si_v.at[NR])
        pltpu.sync_copy(ei_hbm.at[NW * NR + wid, 1], di_v.at[NR])

    plsc.subcore_barrier()

    for b in range(NBUF):
        pltpu.async_copy(hn_sh.at[si_v.at[b]], rows_v.at[b], gsems[b])

    @pl.loop(0, NRMAX // NBUF + 1)
    def _(g):
        for b in range(NBUF):
            j = g * NBUF + b

            @pl.when(j < nrows)
            def _():
                pltpu.make_async_copy(hn_sh.at[pl.ds(0, CHUNK)], rows_v.at[b],
                                      gsems[b]).wait()
                pltpu.sync_copy(rows_v.at[b], s_sh.at[di_v.at[j]], add=True)

            @pl.when(j + NBUF < nrows)
            def _():
                pltpu.async_copy(hn_sh.at[si_v.at[j + NBUF]], rows_v.at[b],
                                 gsems[b])

    plsc.subcore_barrier()
    pltpu.sync_copy(
        s_sh.at[pl.ds(sid * SLICE_N, SLICE_N)],
        s_out.at[pl.ds(cid * NP + sid * SLICE_N, SLICE_N)],
    )


@functools.lru_cache(maxsize=None)
def _sc_kernels():
    mesh = plsc.VectorSubcoreMesh(
        core_axis_name="c", subcore_axis_name="s",
        num_cores=NC, num_subcores=NS,
    )
    params = pltpu.CompilerParams(use_tc_tiling_on_sc=False)
    deg_kernel = pl.kernel(
        _deg_body,
        out_type=jax.ShapeDtypeStruct((NC * NP,), jnp.float32),
        mesh=mesh,
        compiler_params=params,
        scratch_types=[
            pltpu.VMEM((NRMAX, CHUNK), jnp.int32),
            pltpu.VMEM((CHUNK,), jnp.float32),
            pltpu.VMEM((SLICE_N,), jnp.float32),
            pltpu.VMEM_SHARED((NP,), jnp.float32),
            pltpu.SemaphoreType.DMA,
        ],
    )
    scat_kernel = pl.kernel(
        _scat_body,
        out_type=jax.ShapeDtypeStruct((NC * NP, F), jnp.float32),
        mesh=mesh,
        compiler_params=params,
        scratch_types=[
            pltpu.VMEM((NRMAX, CHUNK), jnp.int32),
            pltpu.VMEM((NRMAX, CHUNK), jnp.int32),
            pltpu.VMEM((NBUF, CHUNK, F), jnp.float32),
            pltpu.VMEM((CHUNK, F), jnp.float32),
            pltpu.VMEM_SHARED((NP, F), jnp.float32),
            pltpu.VMEM_SHARED((NP, F), jnp.float32),
        ] + [pltpu.SemaphoreType.DMA] * NBUF,
    )
    return deg_kernel, scat_kernel


BLK = 1024
GRID = NP // BLK


def _mm_body(x_ref, w1_ref, h_ref):
    h_ref[...] = jnp.dot(x_ref[...], w1_ref[...],
                         preferred_element_type=jnp.float32)


def _mm_call(x_p, W1):
    return pl.pallas_call(
        _mm_body,
        grid=(GRID,),
        in_specs=[
            pl.BlockSpec((BLK, D), lambda i: (i, 0)),
            pl.BlockSpec((D, F), lambda i: (0, 0)),
        ],
        out_specs=pl.BlockSpec((BLK, F), lambda i: (i, 0)),
        out_shape=jax.ShapeDtypeStruct((NP, F), jnp.float32),
    )(x_p, W1)


def _hn_body(h_ref, degp_ref, hn_ref, dinv_ref):
    i = pl.program_id(0)
    deg = (degp_ref[pl.ds(i * BLK, BLK)]
           + degp_ref[pl.ds(NP + i * BLK, BLK)] + 1.0)
    dinv = lax.rsqrt(deg)
    hn = h_ref[...] * dinv[:, None]
    hn_ref[...] = jnp.pad(hn, ((0, 0), (0, 128 - F)))
    dinv_ref[0, pl.ds(i * BLK, BLK)] = dinv


def _hn_call(h, degp_flat):
    return pl.pallas_call(
        _hn_body,
        grid=(GRID,),
        in_specs=[
            pl.BlockSpec((BLK, F), lambda i: (i, 0)),
            pl.BlockSpec((NC * NP,), lambda i: (0,)),
        ],
        out_specs=[
            pl.BlockSpec((BLK, 128), lambda i: (i, 0)),
            pl.BlockSpec((1, NP), lambda i: (0, 0)),
        ],
        out_shape=[
            jax.ShapeDtypeStruct((NP, 128), jnp.float32),
            jax.ShapeDtypeStruct((1, NP), jnp.float32),
        ],
    )(h, degp_flat)


def _mlp_body(sp0_ref, sp1_ref, hn_ref, dinv_ref, b1_ref, w2_ref, b2_ref,
              w3_ref, b3_ref, w4_ref, b4_ref, out_ref):
    s = sp0_ref[...] + sp1_ref[...] + hn_ref[..., :F]
    dinv = dinv_ref[0, pl.ds(pl.program_id(0) * BLK, BLK)]
    a = jnp.maximum(s * dinv[:, None] + b1_ref[...], 0.0)
    a = jnp.maximum(
        jnp.dot(a, w2_ref[...], preferred_element_type=jnp.float32)
        + b2_ref[...], 0.0)
    a = jnp.maximum(
        jnp.dot(a, w3_ref[...], preferred_element_type=jnp.float32)
        + b3_ref[...], 0.0)
    z = (jnp.dot(a, w4_ref[...], preferred_element_type=jnp.float32)
         + b4_ref[...])
    out_ref[...] = jax.nn.sigmoid(z)


def _mlp_call(sp, hn, dinv, b1, W2, b2, W3, b3, W4, b4):
    return pl.pallas_call(
        _mlp_body,
        grid=(GRID,),
        in_specs=[
            pl.BlockSpec((BLK, F), lambda i: (i, 0)),
            pl.BlockSpec((BLK, F), lambda i: (i + GRID, 0)),
            pl.BlockSpec((BLK, 128), lambda i: (i, 0)),
            pl.BlockSpec((1, NP), lambda i: (0, 0)),
            pl.BlockSpec((1, F), lambda i: (0, 0)),
            pl.BlockSpec((F, 16), lambda i: (0, 0)),
            pl.BlockSpec((1, 16), lambda i: (0, 0)),
            pl.BlockSpec((16, 32), lambda i: (0, 0)),
            pl.BlockSpec((1, 32), lambda i: (0, 0)),
            pl.BlockSpec((32, 1), lambda i: (0, 0)),
            pl.BlockSpec((1, 1), lambda i: (0, 0)),
        ],
        out_specs=pl.BlockSpec((BLK, 1), lambda i: (i, 0)),
        out_shape=jax.ShapeDtypeStruct((NP, 1), jnp.float32),
    )(sp, sp, hn, dinv, b1, W2, b2, W3, b3, W4, b4)


def kernel(x, edge_index, W1, b1, W2, b2, W3, b3, W4, b4):
    ei3 = edge_index.reshape(2, EROWS, CHUNK).transpose(1, 0, 2)

    deg_kernel, scat_kernel = _sc_kernels()
    degp_flat = deg_kernel(ei3)
    x_p = jnp.pad(x, ((0, NP - N), (0, 0)))
    h = _mm_call(x_p, W1)
    hn_pad, dinv = _hn_call(h, degp_flat)
    sp = scat_kernel(ei3, hn_pad)
    out = _mlp_call(sp, hn_pad, dinv, b1.reshape(1, F), W2, b2.reshape(1, 16),
                    W3, b3.reshape(1, 32), W4, b4.reshape(1, 1))
    return out[:N]
